# baseline (speedup 1.0000x reference)
import functools

import jax
import jax.numpy as jnp
from jax import lax
from jax.experimental import pallas as pl
from jax.experimental.pallas import tpu as pltpu
from jax.experimental.pallas import tpu_sc as plsc

B, T, D = 4096, 77, 768
NW = 32
BPW = B // NW
CH = 16
NSUB = BPW // CH
NG = T * NSUB
NBUF = 4


def _body(xT, tok, pos, out, idx_all, pos_all, *scratch):
    bufs = scratch[:NBUF]
    gsems = scratch[NBUF:2 * NBUF]
    ssems = scratch[2 * NBUF:]
    wid = lax.axis_index("s") * 2 + lax.axis_index("c")
    b0 = wid * BPW

    pltpu.sync_copy(xT.at[:, pl.ds(b0, BPW)], idx_all)
    pltpu.sync_copy(pos, pos_all)

    def idx_ref(t, c):
        return idx_all.at[t, pl.ds(c * CH, CH)]

    def out_ref(t, c):
        return out.at[pl.ds(b0 + c * CH, CH), pl.ds(t, 1)]

    def add_pos(t, buf):
        for h in range(2):
            pv = tuple(pos_all[t, pl.ds(h * 384 + j * 16, 16)]
                       for j in range(24))

            def r_body(r, carry):
                for j in range(24):
                    plsc.addupdate(buf.at[r, 0, pl.ds(h * 384 + j * 16, 16)],
                                   carry[j])
                return carry

            lax.fori_loop(0, CH, r_body, pv)

    pltpu.async_copy(tok.at[idx_ref(0, 0)], bufs[0], gsems[0])
    pltpu.async_copy(tok.at[idx_ref(0, 1)], bufs[1], gsems[1])

    def t_body(t, _):
        for c in range(NSUB):
            s = c % NBUF
            buf, gsem, ssem = bufs[s], gsems[s], ssems[s]
            pltpu.make_async_copy(tok.at[idx_ref(t, c)], buf, gsem).wait()
            add_pos(t, buf)
            pltpu.async_copy(buf, out_ref(t, c), ssem)

            s2 = (c + 2) % NBUF
            buf2, gsem2, ssem2 = bufs[s2], gsems[s2], ssems[s2]
            t2 = t + (c + 2) // NSUB
            c2 = (c + 2) % NSUB
            tp = t - (NSUB + 1 - c) // NSUB
            cp = (c - 2) % NSUB

            @pl.when(t * NSUB + c >= 2)
            def _():
                pltpu.make_async_copy(buf2, out_ref(tp, cp), ssem2).wait()

            @pl.when(t * NSUB + c + 2 < NG)
            def _():
                pltpu.async_copy(tok.at[idx_ref(t2, c2)], buf2, gsem2)
        return 0

    lax.fori_loop(0, T, t_body, 0)
    pltpu.make_async_copy(bufs[2], out_ref(T - 1, NSUB - 2), ssems[2]).wait()
    pltpu.make_async_copy(bufs[3], out_ref(T - 1, NSUB - 1), ssems[3]).wait()


@jax.jit
def kernel(x, tok_embed, pos_embed):
    xT = x.astype(jnp.int32).T
    tok3 = tok_embed.reshape(tok_embed.shape[0], 1, D)
    mesh = plsc.VectorSubcoreMesh(core_axis_name="c", subcore_axis_name="s")
    k = functools.partial(
        pl.kernel,
        mesh=mesh,
        out_type=jax.ShapeDtypeStruct((B, T, D), jnp.float32),
        scratch_types=[
            pltpu.VMEM((T, BPW), jnp.int32),
            pltpu.VMEM((T, D), jnp.float32),
            pltpu.VMEM((CH, 1, D), jnp.float32),
            pltpu.VMEM((CH, 1, D), jnp.float32),
            pltpu.VMEM((CH, 1, D), jnp.float32),
            pltpu.VMEM((CH, 1, D), jnp.float32),
            pltpu.SemaphoreType.DMA,
            pltpu.SemaphoreType.DMA,
            pltpu.SemaphoreType.DMA,
            pltpu.SemaphoreType.DMA,
            pltpu.SemaphoreType.DMA,
            pltpu.SemaphoreType.DMA,
            pltpu.SemaphoreType.DMA,
            pltpu.SemaphoreType.DMA,
        ],
    )(_body)
    return k(xT, tok3, pos_embed)
